# baseline (speedup 1.0000x reference)
import functools

import jax
import jax.numpy as jnp
from jax import lax
from jax.experimental import pallas as pl
from jax.experimental.pallas import tpu as pltpu
from jax.experimental.pallas import tpu_sc as plsc

N = 10000
D = 256
DIN = 128
DOUT = 128
L = 4
NE = 320000
ET = NE + N
NP = 10240
NPT = NP // 16
EP = 335872
NCH = 328
CH = 64
SG = 8
PAD = N
BR = 1024
GRID = NP // BR

_F32 = jnp.float32
_BF16 = jnp.bfloat16
_HIGH = jax.lax.Precision.HIGHEST


def _mm(a, b):
    return jax.lax.dot_general(
        a.astype(_BF16), b.astype(_BF16), (((1,), (0,)), ((), ())),
        preferred_element_type=_F32)


def _att_tail(i, j, xph, asrc_ref, adst_ref, es_ref, ed_ref, mes_ref, med_ref):
    esp = jnp.sum(xph * asrc_ref[...], axis=1)
    edp = jnp.sum(xph * adst_ref[...], axis=1)

    @pl.when(j == 0)
    def _():
        es_ref[...] = esp
        ed_ref[...] = edp

    @pl.when(j == 1)
    def _():
        es_full = es_ref[...] + esp
        ed_full = ed_ref[...] + edp
        es_ref[...] = es_full
        ed_ref[...] = ed_full
        bm_es = jnp.full((1, 128), jnp.max(es_full), _F32)
        bm_ed = jnp.full((1, 128), jnp.max(ed_full), _F32)

        @pl.when(i == 0)
        def _():
            mes_ref[...] = bm_es
            med_ref[...] = bm_ed

        @pl.when(i > 0)
        def _():
            mes_ref[...] = jnp.maximum(mes_ref[...], bm_es)
            med_ref[...] = jnp.maximum(med_ref[...], bm_ed)


def _gates_tail(gates, c_prev):
    ig, fg, gg, og = jnp.split(gates, 4, axis=1)
    cn = jax.nn.sigmoid(fg) * c_prev + jax.nn.sigmoid(ig) * jnp.tanh(gg)
    hn = jax.nn.sigmoid(og) * jnp.tanh(cn)
    return hn, cn


def _xb(lo_ref, hi_ref, den_ref, gb_ref):
    den = den_ref[...] + 1e-16
    on = jnp.concatenate([lo_ref[...], hi_ref[...]], axis=1) / den
    return jnp.tanh(on + gb_ref[...])


_PROJ_OUT = [
    jax.ShapeDtypeStruct((2 * NP, 128), _F32),
    jax.ShapeDtypeStruct((NP,), _F32),
    jax.ShapeDtypeStruct((NP,), _F32),
    jax.ShapeDtypeStruct((1, 128), _F32),
    jax.ShapeDtypeStruct((1, 128), _F32),
]
_PROJ_OUT_SPECS = [
    pl.BlockSpec((BR, 128), lambda i, j: (j * GRID + i, 0)),
    pl.BlockSpec((BR,), lambda i, j: (i,)),
    pl.BlockSpec((BR,), lambda i, j: (i,)),
    pl.BlockSpec((1, 128), lambda i, j: (0, 0)),
    pl.BlockSpec((1, 128), lambda i, j: (0, 0)),
]
_SPEC_GW = pl.BlockSpec((D, 128), lambda i, j: (0, j))
_SPEC_AH = pl.BlockSpec((1, 128), lambda i, j: (0, j))
_SPEC_LO2 = pl.BlockSpec((BR, 128), lambda i, j: (i, 0))
_SPEC_HI2 = pl.BlockSpec((BR, 128), lambda i, j: (GRID + i, 0))
_SPEC_DEN2 = pl.BlockSpec((BR, 1), lambda i, j: (i, 0))
_SPEC_ROW2 = pl.BlockSpec((BR, D), lambda i, j: (i, 0))
_SPEC_WG2 = pl.BlockSpec((D, 4 * D), lambda i, j: (0, 0))
_SPEC_GB2 = pl.BlockSpec((1, D), lambda i, j: (0, 0))


def _f0_body(x_ref, w1_ref, b1_ref, gw_ref, asrc_ref, adst_ref,
             xp_ref, es_ref, ed_ref, mes_ref, med_ref):
    i = pl.program_id(0)
    j = pl.program_id(1)
    x1 = _mm(x_ref[...], w1_ref[...]) + b1_ref[...]
    xph = _mm(x1, gw_ref[...])
    xp_ref[...] = xph
    _att_tail(i, j, xph, asrc_ref, adst_ref, es_ref, ed_ref, mes_ref, med_ref)


def _f0(xpad, w1, b1, gw, asrc, adst):
    return pl.pallas_call(
        _f0_body,
        grid=(GRID, 2),
        in_specs=[
            pl.BlockSpec((BR, DIN), lambda i, j: (i, 0)),
            pl.BlockSpec((DIN, D), lambda i, j: (0, 0)),
            pl.BlockSpec((1, D), lambda i, j: (0, 0)),
            _SPEC_GW, _SPEC_AH, _SPEC_AH,
        ],
        out_specs=_PROJ_OUT_SPECS,
        out_shape=_PROJ_OUT,
    )(xpad, w1, b1, gw, asrc, adst)


def _fm1_body(lo_ref, hi_ref, den_ref, wih_ref, gb_ref, gw_ref,
              asrc_ref, adst_ref,
              hn_ref, cn_ref, xp_ref, es_ref, ed_ref, mes_ref, med_ref):
    i = pl.program_id(0)
    j = pl.program_id(1)

    @pl.when(j == 0)
    def _():
        gates = _mm(_xb(lo_ref, hi_ref, den_ref, gb_ref), wih_ref[...])
        hn, cn = _gates_tail(gates, jnp.zeros_like(cn_ref))
        hn_ref[...] = hn
        cn_ref[...] = cn
    xph = _mm(hn_ref[...], gw_ref[...])
    xp_ref[...] = xph
    _att_tail(i, j, xph, asrc_ref, adst_ref, es_ref, ed_ref, mes_ref, med_ref)


def _fm1(out2, den, wih_t, gb, gw, asrc, adst):
    return pl.pallas_call(
        _fm1_body,
        grid=(GRID, 2),
        in_specs=[_SPEC_LO2, _SPEC_HI2, _SPEC_DEN2, _SPEC_WG2, _SPEC_GB2,
                  _SPEC_GW, _SPEC_AH, _SPEC_AH],
        out_specs=[_SPEC_ROW2, _SPEC_ROW2] + _PROJ_OUT_SPECS,
        out_shape=[
            jax.ShapeDtypeStruct((NP, D), _F32),
            jax.ShapeDtypeStruct((NP, D), _F32),
        ] + _PROJ_OUT,
    )(out2, out2, den, wih_t, gb, gw, asrc, adst)


def _fm_body(lo_ref, hi_ref, den_ref, h_ref, c_ref, wih_ref, whh_ref, gb_ref,
             gw_ref, asrc_ref, adst_ref,
             hn_ref, cn_ref, xp_ref, es_ref, ed_ref, mes_ref, med_ref):
    i = pl.program_id(0)
    j = pl.program_id(1)

    @pl.when(j == 0)
    def _():
        gates = (_mm(_xb(lo_ref, hi_ref, den_ref, gb_ref), wih_ref[...])
                 + _mm(h_ref[...], whh_ref[...]))
        hn, cn = _gates_tail(gates, c_ref[...])
        hn_ref[...] = hn
        cn_ref[...] = cn
    xph = _mm(hn_ref[...], gw_ref[...])
    xp_ref[...] = xph
    _att_tail(i, j, xph, asrc_ref, adst_ref, es_ref, ed_ref, mes_ref, med_ref)


def _fm(out2, den, h, c, wih_t, whh_t, gb, gw, asrc, adst):
    return pl.pallas_call(
        _fm_body,
        grid=(GRID, 2),
        in_specs=[_SPEC_LO2, _SPEC_HI2, _SPEC_DEN2, _SPEC_ROW2, _SPEC_ROW2,
                  _SPEC_WG2, _SPEC_WG2, _SPEC_GB2,
                  _SPEC_GW, _SPEC_AH, _SPEC_AH],
        out_specs=[_SPEC_ROW2, _SPEC_ROW2] + _PROJ_OUT_SPECS,
        out_shape=[
            jax.ShapeDtypeStruct((NP, D), _F32),
            jax.ShapeDtypeStruct((NP, D), _F32),
        ] + _PROJ_OUT,
    )(out2, out2, den, h, c, wih_t, whh_t, gb, gw, asrc, adst)


def _fl_body(lo_ref, hi_ref, den_ref, h_ref, c_ref, wih_ref, whh_ref, gb_ref,
             w2_ref, b2_ref, y_ref):
    gates = (_mm(_xb(lo_ref, hi_ref, den_ref, gb_ref), wih_ref[...])
             + _mm(h_ref[...], whh_ref[...]))
    hn, _ = _gates_tail(gates, c_ref[...])
    y_ref[...] = _mm(hn, w2_ref[...]) + b2_ref[...]


def _fl(out2, den, h, c, wih_t, whh_t, gb, w2, b2):
    return pl.pallas_call(
        _fl_body,
        grid=(GRID,),
        in_specs=[
            pl.BlockSpec((BR, 128), lambda i: (i, 0)),
            pl.BlockSpec((BR, 128), lambda i: (GRID + i, 0)),
            pl.BlockSpec((BR, 1), lambda i: (i, 0)),
            pl.BlockSpec((BR, D), lambda i: (i, 0)),
            pl.BlockSpec((BR, D), lambda i: (i, 0)),
            pl.BlockSpec((D, 4 * D), lambda i: (0, 0)),
            pl.BlockSpec((D, 4 * D), lambda i: (0, 0)),
            pl.BlockSpec((1, D), lambda i: (0, 0)),
            pl.BlockSpec((D, DOUT), lambda i: (0, 0)),
            pl.BlockSpec((1, DOUT), lambda i: (0, 0)),
        ],
        out_specs=pl.BlockSpec((BR, DOUT), lambda i: (i, 0)),
        out_shape=jax.ShapeDtypeStruct((NP, DOUT), _F32),
    )(out2, out2, den, h, c, wih_t, whh_t, gb, w2, b2)


def _sc_edge_body(src_ref, dst_ref, es_ref, ed_ref, mes_ref, med_ref, xp_ref,
                  out_ref, den_ref,
                  es_v, ed_v, sidx_v, didx_v, gidx_v, exk_v, rows_v, m_v,
                  out_acc, den_acc, gsem0, gsem1, dsem, ssem):
    c = lax.axis_index("c")
    s = lax.axis_index("s")

    pltpu.sync_copy(es_ref, es_v)
    pltpu.sync_copy(ed_ref, ed_v)
    pltpu.sync_copy(mes_ref.at[pl.ds(0, 16)], m_v.at[0])
    pltpu.sync_copy(med_ref.at[pl.ds(0, 16)], m_v.at[1])

    neg = jnp.full((16,), -1e30, _F32)
    for k in range((NP - N) // 16):
        es_v[pl.ds(N + k * 16, 16)] = neg
        ed_v[pl.ds(N + k * 16, 16)] = neg

    zv = jnp.zeros((16,), _F32)

    def _zrow(j, _):
        for f in range(8):
            rows_v[0, j, pl.ds(f * 16, 16)] = zv
        return 0
    lax.fori_loop(0, CH, _zrow, 0)
    for g in range(CH // 16):
        exk_v[0, pl.ds(g * 16, 16)] = zv

    for k in range(NPT // CH):
        pltpu.sync_copy(rows_v.at[0], out_acc.at[pl.ds(s * NPT + k * CH, CH)])
    for k in range(NPT // CH):
        pltpu.sync_copy(exk_v.at[0], den_acc.at[pl.ds(s * NPT + k * CH, CH)])
    plsc.subcore_barrier()

    m_sum = m_v[0, pl.ds(0, 16)] + m_v[1, pl.ds(0, 16)]
    big_m = jnp.maximum(m_sum, 0.2 * m_sum)

    offv = jnp.full((16,), c * NP, jnp.int32)

    def _load_sc(sc, sb):
        pltpu.sync_copy(src_ref.at[s, pl.ds(sc * SG, SG)], sidx_v.at[sb])
        pltpu.sync_copy(dst_ref.at[s, pl.ds(sc * SG, SG)], didx_v.at[sb])

    def _issue_gather(ii, tb):
        sb = (ii // SG) % 2
        g = ii % SG
        for gg in range(CH // 16):
            gidx_v[tb, pl.ds(gg * 16, 16)] = (
                sidx_v[sb, g, pl.ds(gg * 16, 16)] + offv)

        @pl.when(tb == 0)
        def _():
            pltpu.async_copy(xp_ref.at[gidx_v.at[0]], rows_v.at[0], gsem0)

        @pl.when(tb == 1)
        def _():
            pltpu.async_copy(xp_ref.at[gidx_v.at[1]], rows_v.at[1], gsem1)

    _load_sc(0, 0)
    _issue_gather(0, 0)

    def _chunk(i, _):
        sb = (i // SG) % 2
        g = i % SG
        b = i % 2

        @pl.when(i >= 2)
        def _():
            pltpu.make_async_copy(es_ref.at[pl.ds(0, CH)], exk_v.at[b],
                                  dsem).wait()
        for gg in range(CH // 16):
            si = sidx_v[sb, g, pl.ds(gg * 16, 16)]
            di = didx_v[sb, g, pl.ds(gg * 16, 16)]
            e = plsc.load_gather(es_v, [si]) + plsc.load_gather(ed_v, [di])
            e = jnp.maximum(e, 0.2 * e)
            exk_v[b, pl.ds(gg * 16, 16)] = jnp.exp(e - big_m)
        pltpu.async_copy(exk_v.at[b], den_acc.at[didx_v.at[sb, g]], dsem,
                         add=True)

        @pl.when(i < NCH - 1)
        def _():
            ip1 = i + 1

            @pl.when(ip1 % SG == 0)
            def _():
                _load_sc(ip1 // SG, (ip1 // SG) % 2)
            @pl.when(i >= 1)
            def _():
                pltpu.make_async_copy(xp_ref.at[pl.ds(0, CH)],
                                      rows_v.at[1 - b], ssem).wait()
            _issue_gather(ip1, 1 - b)

        @pl.when(b == 0)
        def _():
            pltpu.make_async_copy(xp_ref.at[pl.ds(0, CH)], rows_v.at[0],
                                  gsem0).wait()

        @pl.when(b == 1)
        def _():
            pltpu.make_async_copy(xp_ref.at[pl.ds(0, CH)], rows_v.at[1],
                                  gsem1).wait()

        for jg in range(CH // 16):
            w16 = exk_v[b, pl.ds(jg * 16, 16)]
            for t in range(16):
                wv = jnp.full((16,), w16[t], _F32)
                j = jg * 16 + t
                for f in range(8):
                    rows_v[b, j, pl.ds(f * 16, 16)] = (
                        rows_v[b, j, pl.ds(f * 16, 16)] * wv)
        pltpu.async_copy(rows_v.at[b], out_acc.at[didx_v.at[sb, g]], ssem,
                         add=True)
        return 0
    lax.fori_loop(0, NCH, _chunk, 0)
    pltpu.make_async_copy(es_ref.at[pl.ds(0, CH)], exk_v.at[0], dsem).wait()
    pltpu.make_async_copy(es_ref.at[pl.ds(0, CH)], exk_v.at[1], dsem).wait()
    pltpu.make_async_copy(xp_ref.at[pl.ds(0, CH)], rows_v.at[0], ssem).wait()
    plsc.subcore_barrier()

    @pl.when(c == 0)
    def _():
        pltpu.sync_copy(den_acc.at[pl.ds(s * NPT, NPT)],
                        den_ref.at[pl.ds(s * NPT, NPT)])

    for k in range(NPT // 128):
        pltpu.sync_copy(
            out_acc.at[pl.ds(s * NPT + k * 128, 128)],
            out_ref.at[pl.ds(c * NP + s * NPT + k * 128, 128)])


@functools.partial(
    pl.kernel,
    out_type=[
        jax.ShapeDtypeStruct((2 * NP, 128), _F32),
        jax.ShapeDtypeStruct((NP,), _F32),
    ],
    mesh=plsc.VectorSubcoreMesh(core_axis_name="c", subcore_axis_name="s",
                                num_cores=2, num_subcores=16),
    compiler_params=pltpu.CompilerParams(needs_layout_passes=False),
    scratch_types=[
        pltpu.VMEM((NP,), _F32),
        pltpu.VMEM((NP,), _F32),
        pltpu.VMEM((2, SG, CH), jnp.int32),
        pltpu.VMEM((2, SG, CH), jnp.int32),
        pltpu.VMEM((2, CH), jnp.int32),
        pltpu.VMEM((2, CH), _F32),
        pltpu.VMEM((2, CH, 128), _F32),
        pltpu.VMEM((2, 16), _F32),
        pltpu.VMEM_SHARED((NP, 128), _F32),
        pltpu.VMEM_SHARED((NP,), _F32),
        pltpu.SemaphoreType.DMA,
        pltpu.SemaphoreType.DMA,
        pltpu.SemaphoreType.DMA,
        pltpu.SemaphoreType.DMA,
    ],
)
def _sc_edge(src3, dst3, es, ed, mes, med, xp2, out2, den,
             es_v, ed_v, sidx_v, didx_v, gidx_v, exk_v, rows_v, m_v,
             out_acc, den_acc, gsem0, gsem1, dsem, ssem):
    _sc_edge_body(src3, dst3, es, ed, mes, med, xp2, out2, den,
                  es_v, ed_v, sidx_v, didx_v, gidx_v, exk_v, rows_v, m_v,
                  out_acc, den_acc, gsem0, gsem1, dsem, ssem)


def kernel(x, edge_index, lin1_W, lin1_b, gat_W, att_src, att_dst, gat_b,
           lstm_Wih, lstm_Whh, lin2_W, lin2_b):
    loop = jnp.arange(N, dtype=edge_index.dtype)
    src = jnp.concatenate([edge_index[0], loop,
                           jnp.full((EP - ET,), PAD, edge_index.dtype)])
    dst = jnp.concatenate([edge_index[1], loop,
                           jnp.full((EP - ET,), PAD, edge_index.dtype)])
    src3 = src.reshape(16, NCH, CH)
    dst3 = dst.reshape(16, NCH, CH)

    xpad = jnp.zeros((NP, DIN), _F32).at[:N].set(x)

    def _edge(xp2, es, ed, mes, med):
        out2, den = _sc_edge(src3, dst3, es, ed, mes.reshape(128),
                             med.reshape(128), xp2)
        return out2, den.reshape(NP, 1)

    out2, den = _edge(*_f0(xpad, lin1_W, lin1_b.reshape(1, D), gat_W[0],
                           att_src[0].reshape(1, D), att_dst[0].reshape(1, D)))
    h, c, xp2, es, ed, mes, med = _fm1(
        out2, den, lstm_Wih[0].T, gat_b[0].reshape(1, D), gat_W[1],
        att_src[1].reshape(1, D), att_dst[1].reshape(1, D))
    out2, den = _edge(xp2, es, ed, mes, med)
    for i in (1, 2):
        h, c, xp2, es, ed, mes, med = _fm(
            out2, den, h, c, lstm_Wih[i].T, lstm_Whh[i].T,
            gat_b[i].reshape(1, D), gat_W[i + 1],
            att_src[i + 1].reshape(1, D), att_dst[i + 1].reshape(1, D))
        out2, den = _edge(xp2, es, ed, mes, med)
    y = _fl(out2, den, h, c, lstm_Wih[3].T, lstm_Whh[3].T,
            gat_b[3].reshape(1, D), lin2_W, lin2_b.reshape(1, DOUT))
    return y[:N]


# speedup vs baseline: 1.0974x; 1.0974x over previous
import functools

import jax
import jax.numpy as jnp
from jax import lax
from jax.experimental import pallas as pl
from jax.experimental.pallas import tpu as pltpu
from jax.experimental.pallas import tpu_sc as plsc

N = 10000
D = 256
DIN = 128
DOUT = 128
L = 4
NE = 320000
ET = NE + N
NP = 10240
NPT = NP // 16
EP = 335872
NCH = 328
CH = 64
SG = 8
PAD = N
BR = 1024
GRID = NP // BR

_F32 = jnp.float32
_BF16 = jnp.bfloat16
_HIGH = jax.lax.Precision.HIGHEST


def _mm(a, b):
    return jax.lax.dot_general(
        a.astype(_BF16), b.astype(_BF16), (((1,), (0,)), ((), ())),
        preferred_element_type=_F32)


def _lin1_body(x_ref, w_ref, b_ref, o_ref):
    o_ref[...] = _mm(x_ref[...], w_ref[...]) + b_ref[...]


def _lin1(xp, w, b):
    return pl.pallas_call(
        _lin1_body,
        grid=(GRID,),
        in_specs=[
            pl.BlockSpec((BR, DIN), lambda i: (i, 0)),
            pl.BlockSpec((DIN, D), lambda i: (0, 0)),
            pl.BlockSpec((1, D), lambda i: (0, 0)),
        ],
        out_specs=pl.BlockSpec((BR, D), lambda i: (i, 0)),
        out_shape=jax.ShapeDtypeStruct((NP, D), _F32),
    )(xp, w, b)


def _proj_body(x_ref, w_ref, asrc_ref, adst_ref, xp_ref, es_ref, ed_ref,
               mes_ref, med_ref):
    i = pl.program_id(0)
    j = pl.program_id(1)
    xph = _mm(x_ref[...], w_ref[...])
    xp_ref[...] = xph
    esp = jnp.sum(xph * asrc_ref[...], axis=1)
    edp = jnp.sum(xph * adst_ref[...], axis=1)

    @pl.when(j == 0)
    def _():
        es_ref[...] = esp
        ed_ref[...] = edp

    @pl.when(j == 1)
    def _():
        es_full = es_ref[...] + esp
        ed_full = ed_ref[...] + edp
        es_ref[...] = es_full
        ed_ref[...] = ed_full
        bm_es = jnp.full((1, 128), jnp.max(es_full), _F32)
        bm_ed = jnp.full((1, 128), jnp.max(ed_full), _F32)

        @pl.when(i == 0)
        def _():
            mes_ref[...] = bm_es
            med_ref[...] = bm_ed

        @pl.when(i > 0)
        def _():
            mes_ref[...] = jnp.maximum(mes_ref[...], bm_es)
            med_ref[...] = jnp.maximum(med_ref[...], bm_ed)


def _gat_proj(x, w, asrc, adst):
    return pl.pallas_call(
        _proj_body,
        grid=(GRID, 2),
        in_specs=[
            pl.BlockSpec((BR, D), lambda i, j: (i, 0)),
            pl.BlockSpec((D, 128), lambda i, j: (0, j)),
            pl.BlockSpec((1, 128), lambda i, j: (0, j)),
            pl.BlockSpec((1, 128), lambda i, j: (0, j)),
        ],
        out_specs=[
            pl.BlockSpec((BR, 128), lambda i, j: (j * GRID + i, 0)),
            pl.BlockSpec((BR,), lambda i, j: (i,)),
            pl.BlockSpec((BR,), lambda i, j: (i,)),
            pl.BlockSpec((1, 128), lambda i, j: (0, 0)),
            pl.BlockSpec((1, 128), lambda i, j: (0, 0)),
        ],
        out_shape=[
            jax.ShapeDtypeStruct((2 * NP, 128), _F32),
            jax.ShapeDtypeStruct((NP,), _F32),
            jax.ShapeDtypeStruct((NP,), _F32),
            jax.ShapeDtypeStruct((1, 128), _F32),
            jax.ShapeDtypeStruct((1, 128), _F32),
        ],
    )(x, w, asrc, adst)


def _gates_tail(gates, c_prev):
    ig, fg, gg, og = jnp.split(gates, 4, axis=1)
    cn = jax.nn.sigmoid(fg) * c_prev + jax.nn.sigmoid(ig) * jnp.tanh(gg)
    hn = jax.nn.sigmoid(og) * jnp.tanh(cn)
    return hn, cn


def _lstm_body(lo_ref, hi_ref, den_ref, h_ref, c_ref, wih_ref, whh_ref,
               gb_ref, hn_ref, cn_ref):
    den = den_ref[...] + 1e-16
    on = jnp.concatenate([lo_ref[...], hi_ref[...]], axis=1) / den
    xb = jnp.tanh(on + gb_ref[...])
    gates = _mm(xb, wih_ref[...]) + _mm(h_ref[...], whh_ref[...])
    hn, cn = _gates_tail(gates, c_ref[...])
    hn_ref[...] = hn
    cn_ref[...] = cn


def _lstm0_body(lo_ref, hi_ref, den_ref, wih_ref, gb_ref, hn_ref, cn_ref):
    den = den_ref[...] + 1e-16
    on = jnp.concatenate([lo_ref[...], hi_ref[...]], axis=1) / den
    xb = jnp.tanh(on + gb_ref[...])
    gates = _mm(xb, wih_ref[...])
    hn, cn = _gates_tail(gates, jnp.zeros_like(cn_ref))
    hn_ref[...] = hn
    cn_ref[...] = cn


_OUT2 = [
    jax.ShapeDtypeStruct((NP, D), _F32),
    jax.ShapeDtypeStruct((NP, D), _F32),
]
_SPEC_LO = pl.BlockSpec((BR, 128), lambda i: (i, 0))
_SPEC_HI = pl.BlockSpec((BR, 128), lambda i: (GRID + i, 0))
_SPEC_DEN = pl.BlockSpec((BR, 1), lambda i: (i, 0))
_SPEC_ROW = pl.BlockSpec((BR, D), lambda i: (i, 0))
_SPEC_WG = pl.BlockSpec((D, 4 * D), lambda i: (0, 0))
_SPEC_GB = pl.BlockSpec((1, D), lambda i: (0, 0))


def _lstm_step(out2, den, h, c, wih_t, whh_t, gb):
    return pl.pallas_call(
        _lstm_body,
        grid=(GRID,),
        in_specs=[_SPEC_LO, _SPEC_HI, _SPEC_DEN, _SPEC_ROW, _SPEC_ROW,
                  _SPEC_WG, _SPEC_WG, _SPEC_GB],
        out_specs=[_SPEC_ROW, _SPEC_ROW],
        out_shape=_OUT2,
    )(out2, out2, den, h, c, wih_t, whh_t, gb)


def _lstm_step0(out2, den, wih_t, gb):
    return pl.pallas_call(
        _lstm0_body,
        grid=(GRID,),
        in_specs=[_SPEC_LO, _SPEC_HI, _SPEC_DEN, _SPEC_WG, _SPEC_GB],
        out_specs=[_SPEC_ROW, _SPEC_ROW],
        out_shape=_OUT2,
    )(out2, out2, den, wih_t, gb)


def _lin2_body(x_ref, w_ref, b_ref, o_ref):
    o_ref[...] = _mm(x_ref[...], w_ref[...]) + b_ref[...]


def _lin2(h, w, b):
    return pl.pallas_call(
        _lin2_body,
        grid=(GRID,),
        in_specs=[
            pl.BlockSpec((BR, D), lambda i: (i, 0)),
            pl.BlockSpec((D, DOUT), lambda i: (0, 0)),
            pl.BlockSpec((1, DOUT), lambda i: (0, 0)),
        ],
        out_specs=pl.BlockSpec((BR, DOUT), lambda i: (i, 0)),
        out_shape=jax.ShapeDtypeStruct((NP, DOUT), _F32),
    )(h, w, b)


def _sc_edge_body(src_ref, dst_ref, es_ref, ed_ref, mes_ref, med_ref, xp_ref,
                  out_ref, den_ref,
                  es_v, ed_v, sidx_v, didx_v, gidx_v, exk_v, rows_v, m_v,
                  out_acc, den_acc, gsem0, gsem1, dsem, ssem):
    c = lax.axis_index("c")
    s = lax.axis_index("s")

    pltpu.sync_copy(es_ref, es_v)
    pltpu.sync_copy(ed_ref, ed_v)
    pltpu.sync_copy(mes_ref.at[pl.ds(0, 16)], m_v.at[0])
    pltpu.sync_copy(med_ref.at[pl.ds(0, 16)], m_v.at[1])

    neg = jnp.full((16,), -1e30, _F32)
    for k in range((NP - N) // 16):
        es_v[pl.ds(N + k * 16, 16)] = neg
        ed_v[pl.ds(N + k * 16, 16)] = neg

    zv = jnp.zeros((16,), _F32)

    def _zrow(j, _):
        for f in range(8):
            rows_v[0, j, pl.ds(f * 16, 16)] = zv
        return 0
    lax.fori_loop(0, CH, _zrow, 0)
    for g in range(CH // 16):
        exk_v[0, pl.ds(g * 16, 16)] = zv

    for k in range(NPT // CH):
        pltpu.sync_copy(rows_v.at[0], out_acc.at[pl.ds(s * NPT + k * CH, CH)])
    for k in range(NPT // CH):
        pltpu.sync_copy(exk_v.at[0], den_acc.at[pl.ds(s * NPT + k * CH, CH)])
    plsc.subcore_barrier()

    m_sum = m_v[0, pl.ds(0, 16)] + m_v[1, pl.ds(0, 16)]
    big_m = jnp.maximum(m_sum, 0.2 * m_sum)

    offv = jnp.full((16,), c * NP, jnp.int32)

    def _load_sc(sc, sb):
        pltpu.sync_copy(src_ref.at[s, pl.ds(sc * SG, SG)], sidx_v.at[sb])
        pltpu.sync_copy(dst_ref.at[s, pl.ds(sc * SG, SG)], didx_v.at[sb])

    def _issue_gather(ii, tb):
        sb = (ii // SG) % 2
        g = ii % SG
        for gg in range(CH // 16):
            gidx_v[tb, pl.ds(gg * 16, 16)] = (
                sidx_v[sb, g, pl.ds(gg * 16, 16)] + offv)

        @pl.when(tb == 0)
        def _():
            pltpu.async_copy(xp_ref.at[gidx_v.at[0]], rows_v.at[0], gsem0)

        @pl.when(tb == 1)
        def _():
            pltpu.async_copy(xp_ref.at[gidx_v.at[1]], rows_v.at[1], gsem1)

    _load_sc(0, 0)
    _issue_gather(0, 0)

    def _chunk(i, _):
        sb = (i // SG) % 2
        g = i % SG
        b = i % 2

        @pl.when(i >= 2)
        def _():
            pltpu.make_async_copy(es_ref.at[pl.ds(0, CH)], exk_v.at[b],
                                  dsem).wait()
        for gg in range(CH // 16):
            si = sidx_v[sb, g, pl.ds(gg * 16, 16)]
            di = didx_v[sb, g, pl.ds(gg * 16, 16)]
            e = plsc.load_gather(es_v, [si]) + plsc.load_gather(ed_v, [di])
            e = jnp.maximum(e, 0.2 * e)
            exk_v[b, pl.ds(gg * 16, 16)] = jnp.exp(e - big_m)
        pltpu.async_copy(exk_v.at[b], den_acc.at[didx_v.at[sb, g]], dsem,
                         add=True)

        @pl.when(i < NCH - 1)
        def _():
            ip1 = i + 1

            @pl.when(ip1 % SG == 0)
            def _():
                _load_sc(ip1 // SG, (ip1 // SG) % 2)
            @pl.when(i >= 1)
            def _():
                pltpu.make_async_copy(xp_ref.at[pl.ds(0, CH)],
                                      rows_v.at[1 - b], ssem).wait()
            _issue_gather(ip1, 1 - b)

        @pl.when(b == 0)
        def _():
            pltpu.make_async_copy(xp_ref.at[pl.ds(0, CH)], rows_v.at[0],
                                  gsem0).wait()

        @pl.when(b == 1)
        def _():
            pltpu.make_async_copy(xp_ref.at[pl.ds(0, CH)], rows_v.at[1],
                                  gsem1).wait()

        for jg in range(CH // 16):
            w16 = exk_v[b, pl.ds(jg * 16, 16)]
            for t in range(16):
                wv = jnp.full((16,), w16[t], _F32)
                j = jg * 16 + t
                for f in range(8):
                    rows_v[b, j, pl.ds(f * 16, 16)] = (
                        rows_v[b, j, pl.ds(f * 16, 16)] * wv)
        pltpu.async_copy(rows_v.at[b], out_acc.at[didx_v.at[sb, g]], ssem,
                         add=True)
        return 0
    lax.fori_loop(0, NCH, _chunk, 0)
    pltpu.make_async_copy(es_ref.at[pl.ds(0, CH)], exk_v.at[0], dsem).wait()
    pltpu.make_async_copy(es_ref.at[pl.ds(0, CH)], exk_v.at[1], dsem).wait()
    pltpu.make_async_copy(xp_ref.at[pl.ds(0, CH)], rows_v.at[0], ssem).wait()
    plsc.subcore_barrier()

    @pl.when(c == 0)
    def _():
        pltpu.sync_copy(den_acc.at[pl.ds(s * NPT, NPT)],
                        den_ref.at[pl.ds(s * NPT, NPT)])

    for k in range(NPT // 128):
        pltpu.sync_copy(
            out_acc.at[pl.ds(s * NPT + k * 128, 128)],
            out_ref.at[pl.ds(c * NP + s * NPT + k * 128, 128)])


@functools.partial(
    pl.kernel,
    out_type=[
        jax.ShapeDtypeStruct((2 * NP, 128), _F32),
        jax.ShapeDtypeStruct((NP,), _F32),
    ],
    mesh=plsc.VectorSubcoreMesh(core_axis_name="c", subcore_axis_name="s",
                                num_cores=2, num_subcores=16),
    compiler_params=pltpu.CompilerParams(needs_layout_passes=False),
    scratch_types=[
        pltpu.VMEM((NP,), _F32),
        pltpu.VMEM((NP,), _F32),
        pltpu.VMEM((2, SG, CH), jnp.int32),
        pltpu.VMEM((2, SG, CH), jnp.int32),
        pltpu.VMEM((2, CH), jnp.int32),
        pltpu.VMEM((2, CH), _F32),
        pltpu.VMEM((2, CH, 128), _F32),
        pltpu.VMEM((2, 16), _F32),
        pltpu.VMEM_SHARED((NP, 128), _F32),
        pltpu.VMEM_SHARED((NP,), _F32),
        pltpu.SemaphoreType.DMA,
        pltpu.SemaphoreType.DMA,
        pltpu.SemaphoreType.DMA,
        pltpu.SemaphoreType.DMA,
    ],
)
def _sc_edge(src3, dst3, es, ed, mes, med, xp2, out2, den,
             es_v, ed_v, sidx_v, didx_v, gidx_v, exk_v, rows_v, m_v,
             out_acc, den_acc, gsem0, gsem1, dsem, ssem):
    _sc_edge_body(src3, dst3, es, ed, mes, med, xp2, out2, den,
                  es_v, ed_v, sidx_v, didx_v, gidx_v, exk_v, rows_v, m_v,
                  out_acc, den_acc, gsem0, gsem1, dsem, ssem)


def kernel(x, edge_index, lin1_W, lin1_b, gat_W, att_src, att_dst, gat_b,
           lstm_Wih, lstm_Whh, lin2_W, lin2_b):
    loop = jnp.arange(N, dtype=edge_index.dtype)
    src = jnp.concatenate([edge_index[0], loop,
                           jnp.full((EP - ET,), PAD, edge_index.dtype)])
    dst = jnp.concatenate([edge_index[1], loop,
                           jnp.full((EP - ET,), PAD, edge_index.dtype)])
    src3 = src.reshape(16, NCH, CH)
    dst3 = dst.reshape(16, NCH, CH)

    xpad = jnp.zeros((NP, DIN), _F32).at[:N].set(x)
    xcur = _lin1(xpad, lin1_W, lin1_b.reshape(1, D))

    h = c = None
    for i in range(L):
        xp2, es, ed, mes, med = _gat_proj(xcur, gat_W[i],
                                          att_src[i].reshape(1, D),
                                          att_dst[i].reshape(1, D))
        out2, den = _sc_edge(src3, dst3, es, ed, mes.reshape(128),
                             med.reshape(128), xp2)
        den2 = den.reshape(NP, 1)
        wih_t = lstm_Wih[i].T
        gb = gat_b[i].reshape(1, D)
        if i == 0:
            h, c = _lstm_step0(out2, den2, wih_t, gb)
        else:
            h, c = _lstm_step(out2, den2, h, c, wih_t, lstm_Whh[i].T, gb)
        xcur = h

    y = _lin2(xcur, lin2_W, lin2_b.reshape(1, DOUT))
    return y[:N]


# speedup vs baseline: 1.1095x; 1.0110x over previous
import functools

import jax
import jax.numpy as jnp
from jax import lax
from jax.experimental import pallas as pl
from jax.experimental.pallas import tpu as pltpu
from jax.experimental.pallas import tpu_sc as plsc

N = 10000
D = 256
DIN = 128
DOUT = 128
L = 4
NE = 320000
ET = NE + N
NP = 10240
NPT = NP // 16
EP = 335872
NCH = 328
CH = 64
SG = 8
PAD = N
BR = 2048
GRID = NP // BR

_F32 = jnp.float32
_BF16 = jnp.bfloat16
_HIGH = jax.lax.Precision.HIGHEST


def _mm(a, b):
    return jax.lax.dot_general(
        a.astype(_BF16), b.astype(_BF16), (((1,), (0,)), ((), ())),
        preferred_element_type=_F32)


def _lin1_body(x_ref, w_ref, b_ref, o_ref):
    o_ref[...] = _mm(x_ref[...], w_ref[...]) + b_ref[...]


def _lin1(xp, w, b):
    return pl.pallas_call(
        _lin1_body,
        grid=(GRID,),
        in_specs=[
            pl.BlockSpec((BR, DIN), lambda i: (i, 0)),
            pl.BlockSpec((DIN, D), lambda i: (0, 0)),
            pl.BlockSpec((1, D), lambda i: (0, 0)),
        ],
        out_specs=pl.BlockSpec((BR, D), lambda i: (i, 0)),
        out_shape=jax.ShapeDtypeStruct((NP, D), _F32),
    )(xp, w, b)


def _proj_body(x_ref, w_ref, asrc_ref, adst_ref, xp_ref, es_ref, ed_ref,
               mes_ref, med_ref):
    i = pl.program_id(0)
    j = pl.program_id(1)
    xph = _mm(x_ref[...], w_ref[...])
    xp_ref[...] = xph
    esp = jnp.sum(xph * asrc_ref[...], axis=1)
    edp = jnp.sum(xph * adst_ref[...], axis=1)

    @pl.when(j == 0)
    def _():
        es_ref[...] = esp
        ed_ref[...] = edp

    @pl.when(j == 1)
    def _():
        es_full = es_ref[...] + esp
        ed_full = ed_ref[...] + edp
        es_ref[...] = es_full
        ed_ref[...] = ed_full
        bm_es = jnp.full((1, 128), jnp.max(es_full), _F32)
        bm_ed = jnp.full((1, 128), jnp.max(ed_full), _F32)

        @pl.when(i == 0)
        def _():
            mes_ref[...] = bm_es
            med_ref[...] = bm_ed

        @pl.when(i > 0)
        def _():
            mes_ref[...] = jnp.maximum(mes_ref[...], bm_es)
            med_ref[...] = jnp.maximum(med_ref[...], bm_ed)


def _gat_proj(x, w, asrc, adst):
    return pl.pallas_call(
        _proj_body,
        grid=(GRID, 2),
        in_specs=[
            pl.BlockSpec((BR, D), lambda i, j: (i, 0)),
            pl.BlockSpec((D, 128), lambda i, j: (0, j)),
            pl.BlockSpec((1, 128), lambda i, j: (0, j)),
            pl.BlockSpec((1, 128), lambda i, j: (0, j)),
        ],
        out_specs=[
            pl.BlockSpec((BR, 128), lambda i, j: (j * GRID + i, 0)),
            pl.BlockSpec((BR,), lambda i, j: (i,)),
            pl.BlockSpec((BR,), lambda i, j: (i,)),
            pl.BlockSpec((1, 128), lambda i, j: (0, 0)),
            pl.BlockSpec((1, 128), lambda i, j: (0, 0)),
        ],
        out_shape=[
            jax.ShapeDtypeStruct((2 * NP, 128), _F32),
            jax.ShapeDtypeStruct((NP,), _F32),
            jax.ShapeDtypeStruct((NP,), _F32),
            jax.ShapeDtypeStruct((1, 128), _F32),
            jax.ShapeDtypeStruct((1, 128), _F32),
        ],
    )(x, w, asrc, adst)


def _gates_tail(gates, c_prev):
    ig, fg, gg, og = jnp.split(gates, 4, axis=1)
    cn = jax.nn.sigmoid(fg) * c_prev + jax.nn.sigmoid(ig) * jnp.tanh(gg)
    hn = jax.nn.sigmoid(og) * jnp.tanh(cn)
    return hn, cn


def _lstm_body(lo_ref, hi_ref, den_ref, h_ref, c_ref, wih_ref, whh_ref,
               gb_ref, hn_ref, cn_ref):
    den = den_ref[...] + 1e-16
    on = jnp.concatenate([lo_ref[...], hi_ref[...]], axis=1) / den
    xb = jnp.tanh(on + gb_ref[...])
    gates = _mm(xb, wih_ref[...]) + _mm(h_ref[...], whh_ref[...])
    hn, cn = _gates_tail(gates, c_ref[...])
    hn_ref[...] = hn
    cn_ref[...] = cn


def _lstm0_body(lo_ref, hi_ref, den_ref, wih_ref, gb_ref, hn_ref, cn_ref):
    den = den_ref[...] + 1e-16
    on = jnp.concatenate([lo_ref[...], hi_ref[...]], axis=1) / den
    xb = jnp.tanh(on + gb_ref[...])
    gates = _mm(xb, wih_ref[...])
    hn, cn = _gates_tail(gates, jnp.zeros_like(cn_ref))
    hn_ref[...] = hn
    cn_ref[...] = cn


_OUT2 = [
    jax.ShapeDtypeStruct((NP, D), _F32),
    jax.ShapeDtypeStruct((NP, D), _F32),
]
_SPEC_LO = pl.BlockSpec((BR, 128), lambda i: (i, 0))
_SPEC_HI = pl.BlockSpec((BR, 128), lambda i: (GRID + i, 0))
_SPEC_DEN = pl.BlockSpec((BR, 1), lambda i: (i, 0))
_SPEC_ROW = pl.BlockSpec((BR, D), lambda i: (i, 0))
_SPEC_WG = pl.BlockSpec((D, 4 * D), lambda i: (0, 0))
_SPEC_GB = pl.BlockSpec((1, D), lambda i: (0, 0))


def _lstm_step(out2, den, h, c, wih_t, whh_t, gb):
    return pl.pallas_call(
        _lstm_body,
        grid=(GRID,),
        in_specs=[_SPEC_LO, _SPEC_HI, _SPEC_DEN, _SPEC_ROW, _SPEC_ROW,
                  _SPEC_WG, _SPEC_WG, _SPEC_GB],
        out_specs=[_SPEC_ROW, _SPEC_ROW],
        out_shape=_OUT2,
    )(out2, out2, den, h, c, wih_t, whh_t, gb)


def _lstm_step0(out2, den, wih_t, gb):
    return pl.pallas_call(
        _lstm0_body,
        grid=(GRID,),
        in_specs=[_SPEC_LO, _SPEC_HI, _SPEC_DEN, _SPEC_WG, _SPEC_GB],
        out_specs=[_SPEC_ROW, _SPEC_ROW],
        out_shape=_OUT2,
    )(out2, out2, den, wih_t, gb)


def _lin2_body(x_ref, w_ref, b_ref, o_ref):
    o_ref[...] = _mm(x_ref[...], w_ref[...]) + b_ref[...]


def _lin2(h, w, b):
    return pl.pallas_call(
        _lin2_body,
        grid=(GRID,),
        in_specs=[
            pl.BlockSpec((BR, D), lambda i: (i, 0)),
            pl.BlockSpec((D, DOUT), lambda i: (0, 0)),
            pl.BlockSpec((1, DOUT), lambda i: (0, 0)),
        ],
        out_specs=pl.BlockSpec((BR, DOUT), lambda i: (i, 0)),
        out_shape=jax.ShapeDtypeStruct((NP, DOUT), _F32),
    )(h, w, b)


def _sc_edge_body(src_ref, dst_ref, es_ref, ed_ref, mes_ref, med_ref, xp_ref,
                  out_ref, den_ref,
                  es_v, ed_v, sidx_v, didx_v, gidx_v, exk_v, rows_v, m_v,
                  out_acc, den_acc, gsem0, gsem1, dsem, ssem):
    c = lax.axis_index("c")
    s = lax.axis_index("s")

    pltpu.sync_copy(es_ref, es_v)
    pltpu.sync_copy(ed_ref, ed_v)
    pltpu.sync_copy(mes_ref.at[pl.ds(0, 16)], m_v.at[0])
    pltpu.sync_copy(med_ref.at[pl.ds(0, 16)], m_v.at[1])

    neg = jnp.full((16,), -1e30, _F32)
    for k in range((NP - N) // 16):
        es_v[pl.ds(N + k * 16, 16)] = neg
        ed_v[pl.ds(N + k * 16, 16)] = neg

    zv = jnp.zeros((16,), _F32)

    def _zrow(j, _):
        for f in range(8):
            rows_v[0, j, pl.ds(f * 16, 16)] = zv
        return 0
    lax.fori_loop(0, CH, _zrow, 0)
    for g in range(CH // 16):
        exk_v[0, pl.ds(g * 16, 16)] = zv

    for k in range(NPT // CH):
        pltpu.sync_copy(rows_v.at[0], out_acc.at[pl.ds(s * NPT + k * CH, CH)])
    for k in range(NPT // CH):
        pltpu.sync_copy(exk_v.at[0], den_acc.at[pl.ds(s * NPT + k * CH, CH)])
    plsc.subcore_barrier()

    m_sum = m_v[0, pl.ds(0, 16)] + m_v[1, pl.ds(0, 16)]
    big_m = jnp.maximum(m_sum, 0.2 * m_sum)

    offv = jnp.full((16,), c * NP, jnp.int32)

    def _load_sc(sc, sb):
        pltpu.sync_copy(src_ref.at[s, pl.ds(sc * SG, SG)], sidx_v.at[sb])
        pltpu.sync_copy(dst_ref.at[s, pl.ds(sc * SG, SG)], didx_v.at[sb])

    def _issue_gather(ii, tb):
        sb = (ii // SG) % 2
        g = ii % SG
        for gg in range(CH // 16):
            gidx_v[tb, pl.ds(gg * 16, 16)] = (
                sidx_v[sb, g, pl.ds(gg * 16, 16)] + offv)

        @pl.when(tb == 0)
        def _():
            pltpu.async_copy(xp_ref.at[gidx_v.at[0]], rows_v.at[0], gsem0)

        @pl.when(tb == 1)
        def _():
            pltpu.async_copy(xp_ref.at[gidx_v.at[1]], rows_v.at[1], gsem1)

    _load_sc(0, 0)
    _issue_gather(0, 0)

    def _chunk(i, _):
        sb = (i // SG) % 2
        g = i % SG
        b = i % 2

        @pl.when(i >= 2)
        def _():
            pltpu.make_async_copy(es_ref.at[pl.ds(0, CH)], exk_v.at[b],
                                  dsem).wait()
        for gg in range(CH // 16):
            si = sidx_v[sb, g, pl.ds(gg * 16, 16)]
            di = didx_v[sb, g, pl.ds(gg * 16, 16)]
            e = plsc.load_gather(es_v, [si]) + plsc.load_gather(ed_v, [di])
            e = jnp.maximum(e, 0.2 * e)
            exk_v[b, pl.ds(gg * 16, 16)] = jnp.exp(e - big_m)
        pltpu.async_copy(exk_v.at[b], den_acc.at[didx_v.at[sb, g]], dsem,
                         add=True)

        @pl.when(i < NCH - 1)
        def _():
            ip1 = i + 1

            @pl.when(ip1 % SG == 0)
            def _():
                _load_sc(ip1 // SG, (ip1 // SG) % 2)
            @pl.when(i >= 1)
            def _():
                pltpu.make_async_copy(xp_ref.at[pl.ds(0, CH)],
                                      rows_v.at[1 - b], ssem).wait()
            _issue_gather(ip1, 1 - b)

        @pl.when(b == 0)
        def _():
            pltpu.make_async_copy(xp_ref.at[pl.ds(0, CH)], rows_v.at[0],
                                  gsem0).wait()

        @pl.when(b == 1)
        def _():
            pltpu.make_async_copy(xp_ref.at[pl.ds(0, CH)], rows_v.at[1],
                                  gsem1).wait()

        for jg in range(CH // 16):
            w16 = exk_v[b, pl.ds(jg * 16, 16)]
            for t in range(16):
                wv = jnp.full((16,), w16[t], _F32)
                j = jg * 16 + t
                for f in range(8):
                    rows_v[b, j, pl.ds(f * 16, 16)] = (
                        rows_v[b, j, pl.ds(f * 16, 16)] * wv)
        pltpu.async_copy(rows_v.at[b], out_acc.at[didx_v.at[sb, g]], ssem,
                         add=True)
        return 0
    lax.fori_loop(0, NCH, _chunk, 0)
    pltpu.make_async_copy(es_ref.at[pl.ds(0, CH)], exk_v.at[0], dsem).wait()
    pltpu.make_async_copy(es_ref.at[pl.ds(0, CH)], exk_v.at[1], dsem).wait()
    pltpu.make_async_copy(xp_ref.at[pl.ds(0, CH)], rows_v.at[0], ssem).wait()
    plsc.subcore_barrier()

    @pl.when(c == 0)
    def _():
        pltpu.sync_copy(den_acc.at[pl.ds(s * NPT, NPT)],
                        den_ref.at[pl.ds(s * NPT, NPT)])

    for k in range(NPT // 128):
        pltpu.sync_copy(
            out_acc.at[pl.ds(s * NPT + k * 128, 128)],
            out_ref.at[pl.ds(c * NP + s * NPT + k * 128, 128)])


@functools.partial(
    pl.kernel,
    out_type=[
        jax.ShapeDtypeStruct((2 * NP, 128), _F32),
        jax.ShapeDtypeStruct((NP,), _F32),
    ],
    mesh=plsc.VectorSubcoreMesh(core_axis_name="c", subcore_axis_name="s",
                                num_cores=2, num_subcores=16),
    compiler_params=pltpu.CompilerParams(needs_layout_passes=False),
    scratch_types=[
        pltpu.VMEM((NP,), _F32),
        pltpu.VMEM((NP,), _F32),
        pltpu.VMEM((2, SG, CH), jnp.int32),
        pltpu.VMEM((2, SG, CH), jnp.int32),
        pltpu.VMEM((2, CH), jnp.int32),
        pltpu.VMEM((2, CH), _F32),
        pltpu.VMEM((2, CH, 128), _F32),
        pltpu.VMEM((2, 16), _F32),
        pltpu.VMEM_SHARED((NP, 128), _F32),
        pltpu.VMEM_SHARED((NP,), _F32),
        pltpu.SemaphoreType.DMA,
        pltpu.SemaphoreType.DMA,
        pltpu.SemaphoreType.DMA,
        pltpu.SemaphoreType.DMA,
    ],
)
def _sc_edge(src3, dst3, es, ed, mes, med, xp2, out2, den,
             es_v, ed_v, sidx_v, didx_v, gidx_v, exk_v, rows_v, m_v,
             out_acc, den_acc, gsem0, gsem1, dsem, ssem):
    _sc_edge_body(src3, dst3, es, ed, mes, med, xp2, out2, den,
                  es_v, ed_v, sidx_v, didx_v, gidx_v, exk_v, rows_v, m_v,
                  out_acc, den_acc, gsem0, gsem1, dsem, ssem)


def kernel(x, edge_index, lin1_W, lin1_b, gat_W, att_src, att_dst, gat_b,
           lstm_Wih, lstm_Whh, lin2_W, lin2_b):
    loop = jnp.arange(N, dtype=edge_index.dtype)
    src = jnp.concatenate([edge_index[0], loop,
                           jnp.full((EP - ET,), PAD, edge_index.dtype)])
    dst = jnp.concatenate([edge_index[1], loop,
                           jnp.full((EP - ET,), PAD, edge_index.dtype)])
    src3 = src.reshape(16, NCH, CH)
    dst3 = dst.reshape(16, NCH, CH)

    xpad = jnp.zeros((NP, DIN), _F32).at[:N].set(x)
    xcur = _lin1(xpad, lin1_W, lin1_b.reshape(1, D))

    h = c = None
    for i in range(L):
        xp2, es, ed, mes, med = _gat_proj(xcur, gat_W[i],
                                          att_src[i].reshape(1, D),
                                          att_dst[i].reshape(1, D))
        out2, den = _sc_edge(src3, dst3, es, ed, mes.reshape(128),
                             med.reshape(128), xp2)
        den2 = den.reshape(NP, 1)
        wih_t = lstm_Wih[i].T
        gb = gat_b[i].reshape(1, D)
        if i == 0:
            h, c = _lstm_step0(out2, den2, wih_t, gb)
        else:
            h, c = _lstm_step(out2, den2, h, c, wih_t, lstm_Whh[i].T, gb)
        xcur = h

    y = _lin2(xcur, lin2_W, lin2_b.reshape(1, DOUT))
    return y[:N]
